# Initial kernel scaffold; baseline (speedup 1.0000x reference)
import jax
import jax.numpy as jnp
from jax.experimental import pallas as pl


def kernel(x, x_ids, time_gap, cla, Wq, bq, Wk, bk, Wv, bv, theta):
    raise NotImplementedError("write your pallas kernel here")


# speedup vs baseline: 10.7282x; 10.7282x over previous
import functools

import jax
import jax.numpy as jnp
from jax import lax
from jax.experimental import pallas as pl
from jax.experimental.pallas import tpu as pltpu
from jax.experimental.pallas import tpu_sc as plsc

B, S, D, V = 1024, 50, 64, 1000
PAD_ID = 0
INTEREST_ID = 1

_NC, _NS, _L = 2, 16, 16
_NW = _NC * _NS
_RPW = B // _NW
_GROUPS = _RPW // _L

_BB = 256
_INV_SQRT_2PI = 0.3989422804014327


def _tc_weights_body(x_ref, ids_ref, tg_ref, cla_ref, wqt_ref, wkt_ref,
                     bq_ref, bk_ref, wv_ref, th_ref, w_ref):
    fs0, fs1_1, fs1_2, fs2_1, fs2_2 = (th_ref[0], th_ref[1], th_ref[2],
                                       th_ref[3], th_ref[4])
    w_p_g = th_ref[5]
    mu0, sigma0 = th_ref[6], th_ref[7]
    mu1_1, sigma1_1 = th_ref[8], th_ref[9]
    mu1_2, sigma1_2 = th_ref[10], th_ref[11]
    mu2, sigma2, p2, bv0 = th_ref[12], th_ref[13], th_ref[14], th_ref[15]

    x = x_ref[...]
    xb = x.reshape(_BB * S, D)
    q = jnp.dot(xb, wqt_ref[...], preferred_element_type=jnp.float32)
    q = q + bq_ref[...]
    k = jnp.dot(x[:, 0, :], wkt_ref[...], preferred_element_type=jnp.float32)
    k = k + bk_ref[...]
    feats = jnp.tanh(q.reshape(_BB, S, D) + k[:, None, :])
    scores = jnp.sum(feats * wv_ref[0, :][None, None, :], axis=-1) + bv0

    ids = ids_ref[...]
    mask = (ids != PAD_ID) & (ids != INTEREST_ID)
    s_masked = jnp.where(mask, scores, -jnp.inf)
    m = jnp.max(s_masked, axis=-1, keepdims=True)
    e = jnp.where(mask, jnp.exp(s_masked - m), 0.0)
    sm = e / jnp.sum(e, axis=-1, keepdims=True)

    tg = tg_ref[...]
    cl = cla_ref[...]
    pad = jnp.float32(180.0)
    cla0 = jnp.where(cl != 0, pad, tg)
    cla1 = jnp.where(cl != 1, pad, tg)
    cla2 = jnp.where(cl != 2, pad, tg)

    inv_s0 = 1.0 / sigma0
    inv_s11 = 1.0 / sigma1_1
    inv_s12 = 1.0 / sigma1_2
    inv_s2 = 1.0 / sigma2
    pg0 = (fs0 * inv_s0 * _INV_SQRT_2PI) * jnp.exp(
        -0.5 * ((cla0 - mu0) * inv_s0) ** 2)
    pg1 = (fs1_1 * inv_s11 * _INV_SQRT_2PI) * jnp.exp(
        -0.5 * ((cla1 - mu1_1) * inv_s11) ** 2)
    pg1 = pg1 + (fs1_2 * inv_s12 * _INV_SQRT_2PI) * jnp.exp(
        -0.5 * ((cla1 - mu1_2) * inv_s12) ** 2)
    pg2 = (fs2_1 * inv_s2 * _INV_SQRT_2PI) * jnp.exp(
        -0.5 * ((cla2 - mu2) * inv_s2) ** 2)
    pg2 = pg2 + fs2_2 * jnp.exp(p2 * jnp.log(cla2))

    pos = lax.broadcasted_iota(jnp.int32, (_BB, S), 1)
    pg = jnp.where(pos >= 1, pg0 + pg1 + pg2, 0.0)

    w_ref[...] = (1.0 - w_p_g) * sm + w_p_g * pg


def _tc_weights(x, ids, tg_p, cla_p, wqt, wkt, bq2, bk2, wv, th):
    grid = (B // _BB,)
    return pl.pallas_call(
        _tc_weights_body,
        grid=grid,
        in_specs=[
            pl.BlockSpec((_BB, S, D), lambda i: (i, 0, 0)),
            pl.BlockSpec((_BB, S), lambda i: (i, 0)),
            pl.BlockSpec((_BB, S), lambda i: (i, 0)),
            pl.BlockSpec((_BB, S), lambda i: (i, 0)),
            pl.BlockSpec((D, D), lambda i: (0, 0)),
            pl.BlockSpec((D, D), lambda i: (0, 0)),
            pl.BlockSpec((1, D), lambda i: (0, 0)),
            pl.BlockSpec((1, D), lambda i: (0, 0)),
            pl.BlockSpec((1, D), lambda i: (0, 0)),
            pl.BlockSpec(memory_space=pltpu.SMEM),
        ],
        out_specs=pl.BlockSpec((_BB, S), lambda i: (i, 0)),
        out_shape=jax.ShapeDtypeStruct((B, S), jnp.float32),
    )(x, ids, tg_p, cla_p, wqt, wkt, bq2, bk2, wv, th)


def _sc_scatter_body(ids_hbm, w_hbm, out_hbm, ids_v, w_v, acc):
    wid = lax.axis_index("s") * _NC + lax.axis_index("c")
    base = wid * (_RPW * S)
    pltpu.sync_copy(ids_hbm.at[pl.ds(base, _RPW * S)], ids_v)
    pltpu.sync_copy(w_hbm.at[pl.ds(base, _RPW * S)], w_v)

    zeros16 = jnp.zeros((_L,), jnp.float32)

    def _zero(i, c):
        acc[pl.ds(i * _L, _L)] = zeros16
        return c

    lax.fori_loop(0, (_RPW * V) // _L, _zero, 0)

    lane = lax.iota(jnp.int32, _L)
    for g in range(_GROUPS):
        src0 = lane * S + (g * _L * S)
        dst0 = lane * V + (g * _L * V)

        def _scat(s, c, src0=src0, dst0=dst0):
            src = src0 + s
            ids16 = plsc.load_gather(ids_v, [src])
            w16 = plsc.load_gather(w_v, [src])
            plsc.addupdate_scatter(acc, [dst0 + ids16], w16)
            return c

        lax.fori_loop(0, S, _scat, 0)

    pltpu.sync_copy(acc, out_hbm.at[pl.ds(wid * (_RPW * V), _RPW * V)])


@functools.cache
def _sc_scatter():
    return pl.kernel(
        _sc_scatter_body,
        out_type=jax.ShapeDtypeStruct((B * V,), jnp.float32),
        mesh=plsc.VectorSubcoreMesh(core_axis_name="c", subcore_axis_name="s",
                                    num_cores=_NC, num_subcores=_NS),
        scratch_types=[
            pltpu.VMEM((_RPW * S,), jnp.int32),
            pltpu.VMEM((_RPW * S,), jnp.float32),
            pltpu.VMEM((_RPW * V,), jnp.float32),
        ],
        compiler_params=pltpu.CompilerParams(needs_layout_passes=False),
    )


def kernel(x, x_ids, time_gap, cla, Wq, bq, Wk, bk, Wv, bv, theta):
    ids = x_ids.astype(jnp.int32)
    tg_p = jnp.concatenate(
        [jnp.full((B, 1), 1.0, jnp.float32), time_gap.astype(jnp.float32)],
        axis=1)
    cla_p = jnp.concatenate(
        [jnp.full((B, 1), 3, jnp.int32), cla.astype(jnp.int32)], axis=1)
    th = jnp.stack([
        theta["fs0"], theta["fs1_1"], theta["fs1_2"], theta["fs2_1"],
        theta["fs2_2"], theta["w_p_g"], theta["mu0"], theta["sigma0"],
        theta["mu1_1"], theta["sigma1_1"], theta["mu1_2"], theta["sigma1_2"],
        theta["mu2"], theta["sigma2"], theta["p2"], bv[0],
    ]).astype(jnp.float32)

    w = _tc_weights(x, ids, tg_p, cla_p, Wq.T, Wk.T, bq[None, :],
                    bk[None, :], Wv, th)
    out_flat = _sc_scatter()(ids.reshape(-1), w.reshape(-1))
    return out_flat.reshape(B, V)


# speedup vs baseline: 15.9642x; 1.4881x over previous
import functools

import jax
import jax.numpy as jnp
from jax import lax
from jax.experimental import pallas as pl
from jax.experimental.pallas import tpu as pltpu
from jax.experimental.pallas import tpu_sc as plsc

B, S, D, V = 1024, 50, 64, 1000
PAD_ID = 0
INTEREST_ID = 1

_NC, _NS, _L = 2, 16, 16
_NW = _NC * _NS
_RPW = B // _NW
_GROUPS = _RPW // _L
_VPAD = 1024

_BB = 256
_INV_SQRT_2PI = 0.3989422804014327


def _tc_weights_body(xT_ref, idsT_ref, tgT_ref, claT_ref, wq_ref, wk_ref,
                     wv_ref, bq_ref, bk_ref, th_ref, wT_ref, sc_ref):
    fs0, fs1_1, fs1_2, fs2_1, fs2_2 = (th_ref[0], th_ref[1], th_ref[2],
                                       th_ref[3], th_ref[4])
    w_p_g = th_ref[5]
    mu0, sigma0 = th_ref[6], th_ref[7]
    mu1_1, sigma1_1 = th_ref[8], th_ref[9]
    mu1_2, sigma1_2 = th_ref[10], th_ref[11]
    mu2, sigma2, p2, bv0 = th_ref[12], th_ref[13], th_ref[14], th_ref[15]

    wq = wq_ref[...]
    wv = wv_ref[...]
    kT = jnp.dot(wk_ref[...], xT_ref[0],
                 preferred_element_type=jnp.float32) + bk_ref[...]
    for s in range(S):
        q = jnp.dot(wq, xT_ref[s],
                    preferred_element_type=jnp.float32) + bq_ref[...]
        f = jnp.tanh(q + kT)
        sc_ref[pl.ds(s, 1), :] = jnp.dot(
            wv, f, preferred_element_type=jnp.float32) + bv0

    scores = sc_ref[...]
    ids = idsT_ref[...]
    mask = (ids != PAD_ID) & (ids != INTEREST_ID)
    s_masked = jnp.where(mask, scores, -jnp.inf)
    m = jnp.max(s_masked, axis=0, keepdims=True)
    e = jnp.where(mask, jnp.exp(s_masked - m), 0.0)
    sm = e / jnp.sum(e, axis=0, keepdims=True)

    tg = tgT_ref[...]
    cl = claT_ref[...]
    pad = jnp.float32(180.0)
    cla0 = jnp.where(cl != 0, pad, tg)
    cla1 = jnp.where(cl != 1, pad, tg)
    cla2 = jnp.where(cl != 2, pad, tg)

    inv_s0 = 1.0 / sigma0
    inv_s11 = 1.0 / sigma1_1
    inv_s12 = 1.0 / sigma1_2
    inv_s2 = 1.0 / sigma2
    pg = (fs0 * inv_s0 * _INV_SQRT_2PI) * jnp.exp(
        -0.5 * ((cla0 - mu0) * inv_s0) ** 2)
    pg = pg + (fs1_1 * inv_s11 * _INV_SQRT_2PI) * jnp.exp(
        -0.5 * ((cla1 - mu1_1) * inv_s11) ** 2)
    pg = pg + (fs1_2 * inv_s12 * _INV_SQRT_2PI) * jnp.exp(
        -0.5 * ((cla1 - mu1_2) * inv_s12) ** 2)
    pg = pg + (fs2_1 * inv_s2 * _INV_SQRT_2PI) * jnp.exp(
        -0.5 * ((cla2 - mu2) * inv_s2) ** 2)
    pg = pg + fs2_2 * jnp.exp(p2 * jnp.log(cla2))

    pg_full = jnp.concatenate([jnp.zeros((1, _BB), jnp.float32), pg], axis=0)
    wT_ref[...] = (1.0 - w_p_g) * sm + w_p_g * pg_full


def _tc_weights(xT, idsT, tgT, claT, wq, wk, wv, bq2, bk2, th):
    grid = (B // _BB,)
    return pl.pallas_call(
        _tc_weights_body,
        grid=grid,
        in_specs=[
            pl.BlockSpec((S, D, _BB), lambda i: (0, 0, i)),
            pl.BlockSpec((S, _BB), lambda i: (0, i)),
            pl.BlockSpec((S - 1, _BB), lambda i: (0, i)),
            pl.BlockSpec((S - 1, _BB), lambda i: (0, i)),
            pl.BlockSpec((D, D), lambda i: (0, 0)),
            pl.BlockSpec((D, D), lambda i: (0, 0)),
            pl.BlockSpec((1, D), lambda i: (0, 0)),
            pl.BlockSpec((D, 1), lambda i: (0, 0)),
            pl.BlockSpec((D, 1), lambda i: (0, 0)),
            pl.BlockSpec(memory_space=pltpu.SMEM),
        ],
        out_specs=pl.BlockSpec((S, _BB), lambda i: (0, i)),
        out_shape=jax.ShapeDtypeStruct((S, B), jnp.float32),
        scratch_shapes=[pltpu.VMEM((S, _BB), jnp.float32)],
    )(xT, idsT, tgT, claT, wq, wk, wv, bq2, bk2, th)


def _sc_scatter_body(idsT_hbm, wT_hbm, out_hbm, ids_v, w_v, acc, sem):
    wid = lax.axis_index("s") * _NC + lax.axis_index("c")
    col0 = wid * _RPW
    blk0 = (wid // 4) * 128
    my0 = (wid % 4) * _RPW
    c1 = pltpu.async_copy(idsT_hbm.at[:, pl.ds(blk0, 128)], ids_v, sem)
    c2 = pltpu.async_copy(wT_hbm.at[:, pl.ds(blk0, 128)], w_v, sem)

    zeros16 = jnp.zeros((_L,), jnp.float32)

    def _zero(i, c):
        r = i // (_VPAD // _L)
        off = (i % (_VPAD // _L)) * _L
        acc[r, pl.ds(off, _L)] = zeros16
        return c

    lax.fori_loop(0, _RPW * (_VPAD // _L), _zero, 0)
    c1.wait()
    c2.wait()

    lane = lax.iota(jnp.int32, _L)
    for g in range(_GROUPS):
        rows = lane + g * _L

        def _scat(s, c, rows=rows, g=g):
            ids16 = ids_v[s, pl.ds(my0 + g * _L, _L)]
            w16 = w_v[s, pl.ds(my0 + g * _L, _L)]
            plsc.addupdate_scatter(acc, [rows, ids16], w16)
            return c

        lax.fori_loop(0, S, _scat, 0)

    pltpu.sync_copy(acc, out_hbm.at[pl.ds(col0, _RPW), :])


@functools.cache
def _sc_scatter():
    return pl.kernel(
        _sc_scatter_body,
        out_type=jax.ShapeDtypeStruct((B, _VPAD), jnp.float32),
        mesh=plsc.VectorSubcoreMesh(core_axis_name="c", subcore_axis_name="s",
                                    num_cores=_NC, num_subcores=_NS),
        scratch_types=[
            pltpu.VMEM((S, 128), jnp.int32),
            pltpu.VMEM((S, 128), jnp.float32),
            pltpu.VMEM((_RPW, _VPAD), jnp.float32),
            pltpu.SemaphoreType.DMA,
        ],
        compiler_params=pltpu.CompilerParams(needs_layout_passes=False),
    )


def kernel(x, x_ids, time_gap, cla, Wq, bq, Wk, bk, Wv, bv, theta):
    ids = x_ids.astype(jnp.int32)
    xT = jnp.transpose(x, (1, 2, 0))
    idsT = ids.T
    tgT = time_gap.astype(jnp.float32).T
    claT = cla.astype(jnp.int32).T
    th = jnp.stack([
        theta["fs0"], theta["fs1_1"], theta["fs1_2"], theta["fs2_1"],
        theta["fs2_2"], theta["w_p_g"], theta["mu0"], theta["sigma0"],
        theta["mu1_1"], theta["sigma1_1"], theta["mu1_2"], theta["sigma1_2"],
        theta["mu2"], theta["sigma2"], theta["p2"], bv[0],
    ]).astype(jnp.float32)

    wT = _tc_weights(xT, idsT, tgT, claT, Wq, Wk, Wv, bq[:, None],
                     bk[:, None], th)
    return _sc_scatter()(idsT, wT)[:, :V]


# speedup vs baseline: 21.7706x; 1.3637x over previous
import functools

import jax
import jax.numpy as jnp
from jax import lax
from jax.experimental import pallas as pl
from jax.experimental.pallas import tpu as pltpu
from jax.experimental.pallas import tpu_sc as plsc

B, S, D, V = 1024, 50, 64, 1000
PAD_ID = 0
INTEREST_ID = 1

_NC, _NS, _L = 2, 16, 16
_NW = _NC * _NS
_RPW = B // _NW
_GROUPS = _RPW // _L
_VPAD = 1024

_BB = 256
_SG = 5
_INV_SQRT_2PI = 0.3989422804014327


def _tc_weights_body(xT_ref, idsT_ref, tgT_ref, claT_ref, wq_ref, wk_ref,
                     wv_ref, bq_ref, bk_ref, th_ref, wT_ref, sc_ref):
    fs0, fs1_1, fs1_2, fs2_1, fs2_2 = (th_ref[0], th_ref[1], th_ref[2],
                                       th_ref[3], th_ref[4])
    w_p_g = th_ref[5]
    mu0, sigma0 = th_ref[6], th_ref[7]
    mu1_1, sigma1_1 = th_ref[8], th_ref[9]
    mu1_2, sigma1_2 = th_ref[10], th_ref[11]
    mu2, sigma2, p2, bv0 = th_ref[12], th_ref[13], th_ref[14], th_ref[15]

    wq = wq_ref[...]
    wv = wv_ref[...]
    kT = jnp.dot(wk_ref[...], xT_ref[0],
                 preferred_element_type=jnp.float32) + bk_ref[...]
    kT_g = jnp.concatenate([kT] * _SG, axis=1)
    for g0 in range(0, S, _SG):
        xs = jnp.concatenate([xT_ref[g0 + j] for j in range(_SG)], axis=1)
        q = jnp.dot(wq, xs,
                    preferred_element_type=jnp.float32) + bq_ref[...]
        f = jnp.tanh(q + kT_g)
        row = jnp.dot(wv, f, preferred_element_type=jnp.float32) + bv0
        for j in range(_SG):
            sc_ref[pl.ds(g0 + j, 1), :] = row[:, j * _BB:(j + 1) * _BB]

    scores = sc_ref[...]
    ids = idsT_ref[...]
    mask = (ids != PAD_ID) & (ids != INTEREST_ID)
    s_masked = jnp.where(mask, scores, -jnp.inf)
    m = jnp.max(s_masked, axis=0, keepdims=True)
    e = jnp.where(mask, jnp.exp(s_masked - m), 0.0)
    sm = e / jnp.sum(e, axis=0, keepdims=True)

    tg = tgT_ref[...]
    cl = claT_ref[...]
    pad = jnp.float32(180.0)
    cla0 = jnp.where(cl != 0, pad, tg)
    cla1 = jnp.where(cl != 1, pad, tg)
    cla2 = jnp.where(cl != 2, pad, tg)

    inv_s0 = 1.0 / sigma0
    inv_s11 = 1.0 / sigma1_1
    inv_s12 = 1.0 / sigma1_2
    inv_s2 = 1.0 / sigma2
    pg = (fs0 * inv_s0 * _INV_SQRT_2PI) * jnp.exp(
        -0.5 * ((cla0 - mu0) * inv_s0) ** 2)
    pg = pg + (fs1_1 * inv_s11 * _INV_SQRT_2PI) * jnp.exp(
        -0.5 * ((cla1 - mu1_1) * inv_s11) ** 2)
    pg = pg + (fs1_2 * inv_s12 * _INV_SQRT_2PI) * jnp.exp(
        -0.5 * ((cla1 - mu1_2) * inv_s12) ** 2)
    pg = pg + (fs2_1 * inv_s2 * _INV_SQRT_2PI) * jnp.exp(
        -0.5 * ((cla2 - mu2) * inv_s2) ** 2)
    pg = pg + fs2_2 * jnp.exp(p2 * jnp.log(cla2))

    pg_full = jnp.concatenate([jnp.zeros((1, _BB), jnp.float32), pg], axis=0)
    wT_ref[...] = (1.0 - w_p_g) * sm + w_p_g * pg_full


def _tc_weights(xT, idsT, tgT, claT, wq, wk, wv, bq2, bk2, th):
    grid = (B // _BB,)
    return pl.pallas_call(
        _tc_weights_body,
        grid=grid,
        in_specs=[
            pl.BlockSpec((S, D, _BB), lambda i: (0, 0, i)),
            pl.BlockSpec((S, _BB), lambda i: (0, i)),
            pl.BlockSpec((S - 1, _BB), lambda i: (0, i)),
            pl.BlockSpec((S - 1, _BB), lambda i: (0, i)),
            pl.BlockSpec((D, D), lambda i: (0, 0)),
            pl.BlockSpec((D, D), lambda i: (0, 0)),
            pl.BlockSpec((1, D), lambda i: (0, 0)),
            pl.BlockSpec((D, 1), lambda i: (0, 0)),
            pl.BlockSpec((D, 1), lambda i: (0, 0)),
            pl.BlockSpec(memory_space=pltpu.SMEM),
        ],
        out_specs=pl.BlockSpec((S, _BB), lambda i: (0, i)),
        out_shape=jax.ShapeDtypeStruct((S, B), jnp.float32),
        scratch_shapes=[pltpu.VMEM((S, _BB), jnp.float32)],
    )(xT, idsT, tgT, claT, wq, wk, wv, bq2, bk2, th)


def _sc_scatter_body(idsT_hbm, wT_hbm, out_hbm, ids_v, w_v, acc, sem):
    wid = lax.axis_index("s") * _NC + lax.axis_index("c")
    col0 = wid * _RPW
    blk0 = (wid // 4) * 128
    my0 = (wid % 4) * _RPW
    c1 = pltpu.async_copy(idsT_hbm.at[:, pl.ds(blk0, 128)], ids_v, sem)
    c2 = pltpu.async_copy(wT_hbm.at[:, pl.ds(blk0, 128)], w_v, sem)

    zeros16 = jnp.zeros((_L,), jnp.float32)

    def _zero(i, c):
        r = i // (_VPAD // _L)
        off = (i % (_VPAD // _L)) * _L
        acc[r, pl.ds(off, _L)] = zeros16
        return c

    lax.fori_loop(0, _RPW * (_VPAD // _L), _zero, 0)
    c1.wait()
    c2.wait()

    lane = lax.iota(jnp.int32, _L)
    for g in range(_GROUPS):
        rows = lane + g * _L

        def _scat(s, c, rows=rows, g=g):
            ids16 = ids_v[s, pl.ds(my0 + g * _L, _L)]
            w16 = w_v[s, pl.ds(my0 + g * _L, _L)]
            plsc.addupdate_scatter(acc, [rows, ids16], w16)
            return c

        lax.fori_loop(0, S, _scat, 0)

    pltpu.sync_copy(acc, out_hbm.at[pl.ds(col0, _RPW), :])


@functools.cache
def _sc_scatter():
    return pl.kernel(
        _sc_scatter_body,
        out_type=jax.ShapeDtypeStruct((B, _VPAD), jnp.float32),
        mesh=plsc.VectorSubcoreMesh(core_axis_name="c", subcore_axis_name="s",
                                    num_cores=_NC, num_subcores=_NS),
        scratch_types=[
            pltpu.VMEM((S, 128), jnp.int32),
            pltpu.VMEM((S, 128), jnp.float32),
            pltpu.VMEM((_RPW, _VPAD), jnp.float32),
            pltpu.SemaphoreType.DMA,
        ],
        compiler_params=pltpu.CompilerParams(needs_layout_passes=False),
    )


def kernel(x, x_ids, time_gap, cla, Wq, bq, Wk, bk, Wv, bv, theta):
    ids = x_ids.astype(jnp.int32)
    xT = jnp.transpose(x, (1, 2, 0))
    idsT = ids.T
    tgT = time_gap.astype(jnp.float32).T
    claT = cla.astype(jnp.int32).T
    th = jnp.stack([
        theta["fs0"], theta["fs1_1"], theta["fs1_2"], theta["fs2_1"],
        theta["fs2_2"], theta["w_p_g"], theta["mu0"], theta["sigma0"],
        theta["mu1_1"], theta["sigma1_1"], theta["mu1_2"], theta["sigma1_2"],
        theta["mu2"], theta["sigma2"], theta["p2"], bv[0],
    ]).astype(jnp.float32)

    wT = _tc_weights(xT, idsT, tgT, claT, Wq, Wk, Wv, bq[:, None],
                     bk[:, None], th)
    return _sc_scatter()(idsT, wT)[:, :V]


# speedup vs baseline: 25.7867x; 1.1845x over previous
import functools

import jax
import jax.numpy as jnp
from jax import lax
from jax.experimental import pallas as pl
from jax.experimental.pallas import tpu as pltpu
from jax.experimental.pallas import tpu_sc as plsc

B, S, D, V = 1024, 50, 64, 1000
PAD_ID = 0
INTEREST_ID = 1

_NC, _NS, _L = 2, 16, 16
_NW = _NC * _NS
_RPW = B // _NW
_GROUPS = _RPW // _L
_VPAD = 1024

_BB = 256
_SG = 5
_INV_SQRT_2PI = 0.3989422804014327


def _tc_weights_body(xT_ref, idsT_ref, tgT_ref, claT_ref, wq_ref, wk_ref,
                     wv_ref, bq_ref, bk_ref, th_ref, wT_ref, sc_ref):
    fs0, fs1_1, fs1_2, fs2_1, fs2_2 = (th_ref[0], th_ref[1], th_ref[2],
                                       th_ref[3], th_ref[4])
    w_p_g = th_ref[5]
    mu0, sigma0 = th_ref[6], th_ref[7]
    mu1_1, sigma1_1 = th_ref[8], th_ref[9]
    mu1_2, sigma1_2 = th_ref[10], th_ref[11]
    mu2, sigma2, p2, bv0 = th_ref[12], th_ref[13], th_ref[14], th_ref[15]

    wq = wq_ref[...]
    wv = wv_ref[...]
    bias_col = jnp.transpose(bq_ref[...] + bk_ref[...], (1, 0))
    kT = jnp.dot(wk_ref[...], xT_ref[0],
                 preferred_element_type=jnp.float32) + bias_col
    kT_g = jnp.concatenate([kT] * _SG, axis=1)
    for g0 in range(0, S, _SG):
        xs = jnp.concatenate([xT_ref[g0 + j] for j in range(_SG)], axis=1)
        q = jnp.dot(wq, xs, preferred_element_type=jnp.float32)
        f = jnp.tanh(q + kT_g)
        row = jnp.dot(wv, f, preferred_element_type=jnp.float32) + bv0
        for j in range(_SG):
            sc_ref[pl.ds(g0 + j, 1), :] = row[:, j * _BB:(j + 1) * _BB]

    scores = sc_ref[...]
    ids = idsT_ref[...]
    mask = (ids != PAD_ID) & (ids != INTEREST_ID)
    s_masked = jnp.where(mask, scores, -jnp.inf)
    m = jnp.max(s_masked, axis=0, keepdims=True)
    e = jnp.where(mask, jnp.exp(s_masked - m), 0.0)
    sm = e / jnp.sum(e, axis=0, keepdims=True)

    tg = tgT_ref[...]
    cl = claT_ref[...]
    pad = jnp.float32(180.0)
    cla0 = jnp.where(cl != 0, pad, tg)
    cla1 = jnp.where(cl != 1, pad, tg)
    cla2 = jnp.where(cl != 2, pad, tg)

    inv_s0 = 1.0 / sigma0
    inv_s11 = 1.0 / sigma1_1
    inv_s12 = 1.0 / sigma1_2
    inv_s2 = 1.0 / sigma2
    pg = (fs0 * inv_s0 * _INV_SQRT_2PI) * jnp.exp(
        -0.5 * ((cla0 - mu0) * inv_s0) ** 2)
    pg = pg + (fs1_1 * inv_s11 * _INV_SQRT_2PI) * jnp.exp(
        -0.5 * ((cla1 - mu1_1) * inv_s11) ** 2)
    pg = pg + (fs1_2 * inv_s12 * _INV_SQRT_2PI) * jnp.exp(
        -0.5 * ((cla1 - mu1_2) * inv_s12) ** 2)
    pg = pg + (fs2_1 * inv_s2 * _INV_SQRT_2PI) * jnp.exp(
        -0.5 * ((cla2 - mu2) * inv_s2) ** 2)
    pg = pg + fs2_2 * jnp.exp(p2 * jnp.log(cla2))

    pg_full = jnp.concatenate([jnp.zeros((1, _BB), jnp.float32), pg], axis=0)
    wT_ref[...] = (1.0 - w_p_g) * sm + w_p_g * pg_full


def _tc_weights(xT, idsT, tgT, claT, wq, wk, wv, bq2, bk2, th):
    grid = (B // _BB,)
    return pl.pallas_call(
        _tc_weights_body,
        grid=grid,
        in_specs=[
            pl.BlockSpec((S, D, _BB), lambda i: (0, 0, i)),
            pl.BlockSpec((S, _BB), lambda i: (0, i)),
            pl.BlockSpec((S - 1, _BB), lambda i: (0, i)),
            pl.BlockSpec((S - 1, _BB), lambda i: (0, i)),
            pl.BlockSpec((D, D), lambda i: (0, 0)),
            pl.BlockSpec((D, D), lambda i: (0, 0)),
            pl.BlockSpec((1, D), lambda i: (0, 0)),
            pl.BlockSpec((1, D), lambda i: (0, 0)),
            pl.BlockSpec((1, D), lambda i: (0, 0)),
            pl.BlockSpec(memory_space=pltpu.SMEM),
        ],
        out_specs=pl.BlockSpec((S, _BB), lambda i: (0, i)),
        out_shape=jax.ShapeDtypeStruct((S, B), jnp.float32),
        scratch_shapes=[pltpu.VMEM((S, _BB), jnp.float32)],
    )(xT, idsT, tgT, claT, wq, wk, wv, bq2, bk2, th)


def _sc_scatter_body(idsT_hbm, wT_hbm, out_hbm, ids_v, w_v, acc, sem):
    wid = lax.axis_index("s") * _NC + lax.axis_index("c")
    col0 = wid * _RPW
    blk0 = (wid // 4) * 128
    my0 = (wid % 4) * _RPW
    c1 = pltpu.async_copy(idsT_hbm.at[:, pl.ds(blk0, 128)], ids_v, sem)
    c2 = pltpu.async_copy(wT_hbm.at[:, pl.ds(blk0, 128)], w_v, sem)

    zeros16 = jnp.zeros((_L,), jnp.float32)

    def _zero(i, c):
        r = i // 8
        off = (i % 8) * 128
        for u in range(8):
            acc[r, pl.ds(off + u * _L, _L)] = zeros16
        return c

    lax.fori_loop(0, _RPW * _VPAD // (8 * _L), _zero, 0)
    c1.wait()
    c2.wait()

    lane = lax.iota(jnp.int32, _L)
    for g in range(_GROUPS):
        rows = lane + g * _L

        def _scat(s, c, rows=rows, g=g):
            ids16 = ids_v[s, pl.ds(my0 + g * _L, _L)]
            w16 = w_v[s, pl.ds(my0 + g * _L, _L)]
            plsc.addupdate_scatter(acc, [rows, ids16], w16)
            return c

        lax.fori_loop(0, S, _scat, 0)

    pltpu.sync_copy(acc, out_hbm.at[pl.ds(col0, _RPW), :])


@functools.cache
def _sc_scatter():
    return pl.kernel(
        _sc_scatter_body,
        out_type=jax.ShapeDtypeStruct((B, _VPAD), jnp.float32),
        mesh=plsc.VectorSubcoreMesh(core_axis_name="c", subcore_axis_name="s",
                                    num_cores=_NC, num_subcores=_NS),
        scratch_types=[
            pltpu.VMEM((S, 128), jnp.int32),
            pltpu.VMEM((S, 128), jnp.float32),
            pltpu.VMEM((_RPW, _VPAD), jnp.float32),
            pltpu.SemaphoreType.DMA,
        ],
        compiler_params=pltpu.CompilerParams(needs_layout_passes=False),
    )


def kernel(x, x_ids, time_gap, cla, Wq, bq, Wk, bk, Wv, bv, theta):
    ids = x_ids.astype(jnp.int32)
    xT = jnp.transpose(x, (1, 2, 0))
    idsT = ids.T
    tgT = time_gap.astype(jnp.float32).T
    claT = cla.astype(jnp.int32).T
    th = jnp.stack([
        theta["fs0"], theta["fs1_1"], theta["fs1_2"], theta["fs2_1"],
        theta["fs2_2"], theta["w_p_g"], theta["mu0"], theta["sigma0"],
        theta["mu1_1"], theta["sigma1_1"], theta["mu1_2"], theta["sigma1_2"],
        theta["mu2"], theta["sigma2"], theta["p2"], bv[0],
    ]).astype(jnp.float32)

    wT = _tc_weights(xT, idsT, tgT, claT, Wq, Wk, Wv, bq[None, :],
                     bk[None, :], th)
    return _sc_scatter()(idsT, wT)[:, :V]


# speedup vs baseline: 27.8650x; 1.0806x over previous
import functools

import jax
import jax.numpy as jnp
from jax import lax
from jax.experimental import pallas as pl
from jax.experimental.pallas import tpu as pltpu
from jax.experimental.pallas import tpu_sc as plsc

B, S, D, V = 1024, 50, 64, 1000
PAD_ID = 0
INTEREST_ID = 1

_NC, _NS, _L = 2, 16, 16
_NW = _NC * _NS
_RPW = B // _NW
_GROUPS = _RPW // _L
_VPAD = 1024

_BB = 256
_SG = 25
_INV_SQRT_2PI = 0.3989422804014327


def _tc_weights_body(xT_ref, idsT_ref, tgT_ref, claT_ref, wq_ref, wk_ref,
                     wv_ref, bq_ref, bk_ref, th_ref, wT_ref, sc_ref):
    fs0, fs1_1, fs1_2, fs2_1, fs2_2 = (th_ref[0], th_ref[1], th_ref[2],
                                       th_ref[3], th_ref[4])
    w_p_g = th_ref[5]
    mu0, sigma0 = th_ref[6], th_ref[7]
    mu1_1, sigma1_1 = th_ref[8], th_ref[9]
    mu1_2, sigma1_2 = th_ref[10], th_ref[11]
    mu2, sigma2, p2, bv0 = th_ref[12], th_ref[13], th_ref[14], th_ref[15]

    wq = wq_ref[...]
    wv = wv_ref[...]
    bias_col = jnp.transpose(bq_ref[...] + bk_ref[...], (1, 0))
    kT = jnp.dot(wk_ref[...], xT_ref[0],
                 preferred_element_type=jnp.float32) + bias_col
    kT_g = jnp.concatenate([kT] * _SG, axis=1)
    for g0 in range(0, S, _SG):
        xs = jnp.concatenate([xT_ref[g0 + j] for j in range(_SG)], axis=1)
        q = jnp.dot(wq, xs, preferred_element_type=jnp.float32)
        f = jnp.tanh(q + kT_g)
        row = jnp.dot(wv, f, preferred_element_type=jnp.float32) + bv0
        for j in range(_SG):
            sc_ref[pl.ds(g0 + j, 1), :] = row[:, j * _BB:(j + 1) * _BB]

    scores = sc_ref[...]
    ids = idsT_ref[...]
    mask = (ids != PAD_ID) & (ids != INTEREST_ID)
    s_masked = jnp.where(mask, scores, -jnp.inf)
    m = jnp.max(s_masked, axis=0, keepdims=True)
    e = jnp.where(mask, jnp.exp(s_masked - m), 0.0)
    sm = e / jnp.sum(e, axis=0, keepdims=True)

    tg = tgT_ref[...]
    cl = claT_ref[...]
    pad = jnp.float32(180.0)
    cla0 = jnp.where(cl != 0, pad, tg)
    cla1 = jnp.where(cl != 1, pad, tg)
    cla2 = jnp.where(cl != 2, pad, tg)

    inv_s0 = 1.0 / sigma0
    inv_s11 = 1.0 / sigma1_1
    inv_s12 = 1.0 / sigma1_2
    inv_s2 = 1.0 / sigma2
    pg = (fs0 * inv_s0 * _INV_SQRT_2PI) * jnp.exp(
        -0.5 * ((cla0 - mu0) * inv_s0) ** 2)
    pg = pg + (fs1_1 * inv_s11 * _INV_SQRT_2PI) * jnp.exp(
        -0.5 * ((cla1 - mu1_1) * inv_s11) ** 2)
    pg = pg + (fs1_2 * inv_s12 * _INV_SQRT_2PI) * jnp.exp(
        -0.5 * ((cla1 - mu1_2) * inv_s12) ** 2)
    pg = pg + (fs2_1 * inv_s2 * _INV_SQRT_2PI) * jnp.exp(
        -0.5 * ((cla2 - mu2) * inv_s2) ** 2)
    pg = pg + fs2_2 * jnp.exp(p2 * jnp.log(cla2))

    pg_full = jnp.concatenate([jnp.zeros((1, _BB), jnp.float32), pg], axis=0)
    wT_ref[...] = (1.0 - w_p_g) * sm + w_p_g * pg_full


def _tc_weights(xT, idsT, tgT, claT, wq, wk, wv, bq2, bk2, th):
    grid = (B // _BB,)
    return pl.pallas_call(
        _tc_weights_body,
        grid=grid,
        in_specs=[
            pl.BlockSpec((S, D, _BB), lambda i: (0, 0, i)),
            pl.BlockSpec((S, _BB), lambda i: (0, i)),
            pl.BlockSpec((S - 1, _BB), lambda i: (0, i)),
            pl.BlockSpec((S - 1, _BB), lambda i: (0, i)),
            pl.BlockSpec((D, D), lambda i: (0, 0)),
            pl.BlockSpec((D, D), lambda i: (0, 0)),
            pl.BlockSpec((1, D), lambda i: (0, 0)),
            pl.BlockSpec((1, D), lambda i: (0, 0)),
            pl.BlockSpec((1, D), lambda i: (0, 0)),
            pl.BlockSpec(memory_space=pltpu.SMEM),
        ],
        out_specs=pl.BlockSpec((S, _BB), lambda i: (0, i)),
        out_shape=jax.ShapeDtypeStruct((S, B), jnp.float32),
        scratch_shapes=[pltpu.VMEM((S, _BB), jnp.float32)],
    )(xT, idsT, tgT, claT, wq, wk, wv, bq2, bk2, th)


def _sc_scatter_body(idsT_hbm, wT_hbm, out_hbm, ids_v, w_v, acc, sem):
    wid = lax.axis_index("s") * _NC + lax.axis_index("c")
    col0 = wid * _RPW
    blk0 = (wid // 4) * 128
    my0 = (wid % 4) * _RPW
    c1 = pltpu.async_copy(idsT_hbm.at[:, pl.ds(blk0, 128)], ids_v, sem)
    c2 = pltpu.async_copy(wT_hbm.at[:, pl.ds(blk0, 128)], w_v, sem)

    zeros16 = jnp.zeros((_L,), jnp.float32)

    def _zero(i, c):
        r = i // 8
        off = (i % 8) * 128
        for u in range(8):
            acc[r, pl.ds(off + u * _L, _L)] = zeros16
        return c

    lax.fori_loop(0, _RPW * _VPAD // (8 * _L), _zero, 0)
    c1.wait()
    c2.wait()

    lane = lax.iota(jnp.int32, _L)
    for g in range(_GROUPS):
        rows = lane + g * _L

        def _scat(s, c, rows=rows, g=g):
            ids16 = ids_v[s, pl.ds(my0 + g * _L, _L)]
            w16 = w_v[s, pl.ds(my0 + g * _L, _L)]
            plsc.addupdate_scatter(acc, [rows, ids16], w16)
            return c

        lax.fori_loop(0, S, _scat, 0)

    pltpu.sync_copy(acc, out_hbm.at[pl.ds(col0, _RPW), :])


@functools.cache
def _sc_scatter():
    return pl.kernel(
        _sc_scatter_body,
        out_type=jax.ShapeDtypeStruct((B, _VPAD), jnp.float32),
        mesh=plsc.VectorSubcoreMesh(core_axis_name="c", subcore_axis_name="s",
                                    num_cores=_NC, num_subcores=_NS),
        scratch_types=[
            pltpu.VMEM((S, 128), jnp.int32),
            pltpu.VMEM((S, 128), jnp.float32),
            pltpu.VMEM((_RPW, _VPAD), jnp.float32),
            pltpu.SemaphoreType.DMA,
        ],
        compiler_params=pltpu.CompilerParams(needs_layout_passes=False),
    )


def kernel(x, x_ids, time_gap, cla, Wq, bq, Wk, bk, Wv, bv, theta):
    ids = x_ids.astype(jnp.int32)
    xT = jnp.transpose(x, (1, 2, 0))
    idsT = ids.T
    tgT = time_gap.astype(jnp.float32).T
    claT = cla.astype(jnp.int32).T
    th = jnp.stack([
        theta["fs0"], theta["fs1_1"], theta["fs1_2"], theta["fs2_1"],
        theta["fs2_2"], theta["w_p_g"], theta["mu0"], theta["sigma0"],
        theta["mu1_1"], theta["sigma1_1"], theta["mu1_2"], theta["sigma1_2"],
        theta["mu2"], theta["sigma2"], theta["p2"], bv[0],
    ]).astype(jnp.float32)

    wT = _tc_weights(xT, idsT, tgT, claT, Wq, Wk, Wv, bq[None, :],
                     bk[None, :], th)
    return _sc_scatter()(idsT, wT)[:, :V]


# speedup vs baseline: 28.0034x; 1.0050x over previous
import functools

import jax
import jax.numpy as jnp
from jax import lax
from jax.experimental import pallas as pl
from jax.experimental.pallas import tpu as pltpu
from jax.experimental.pallas import tpu_sc as plsc

B, S, D, V = 1024, 50, 64, 1000
PAD_ID = 0
INTEREST_ID = 1

_NC, _NS, _L = 2, 16, 16
_NW = _NC * _NS
_RPW = B // _NW
_GROUPS = _RPW // _L
_VPAD = 1024

_BB = 256
_SG = 50
_INV_SQRT_2PI = 0.3989422804014327


def _tc_weights_body(xT_ref, idsT_ref, tgT_ref, claT_ref, wq_ref, wk_ref,
                     wv_ref, bq_ref, bk_ref, th_ref, wT_ref, sc_ref):
    fs0, fs1_1, fs1_2, fs2_1, fs2_2 = (th_ref[0], th_ref[1], th_ref[2],
                                       th_ref[3], th_ref[4])
    w_p_g = th_ref[5]
    mu0, sigma0 = th_ref[6], th_ref[7]
    mu1_1, sigma1_1 = th_ref[8], th_ref[9]
    mu1_2, sigma1_2 = th_ref[10], th_ref[11]
    mu2, sigma2, p2, bv0 = th_ref[12], th_ref[13], th_ref[14], th_ref[15]

    wq = wq_ref[...]
    wv = wv_ref[...]
    bias_col = jnp.transpose(bq_ref[...] + bk_ref[...], (1, 0))
    kT = jnp.dot(wk_ref[...], xT_ref[0],
                 preferred_element_type=jnp.float32) + bias_col
    kT_g = jnp.concatenate([kT] * _SG, axis=1)
    for g0 in range(0, S, _SG):
        xs = jnp.concatenate([xT_ref[g0 + j] for j in range(_SG)], axis=1)
        q = jnp.dot(wq, xs, preferred_element_type=jnp.float32)
        f = jnp.tanh(q + kT_g)
        row = jnp.dot(wv, f, preferred_element_type=jnp.float32) + bv0
        for j in range(_SG):
            sc_ref[pl.ds(g0 + j, 1), :] = row[:, j * _BB:(j + 1) * _BB]

    scores = sc_ref[...]
    ids = idsT_ref[...]
    mask = (ids != PAD_ID) & (ids != INTEREST_ID)
    s_masked = jnp.where(mask, scores, -jnp.inf)
    m = jnp.max(s_masked, axis=0, keepdims=True)
    e = jnp.where(mask, jnp.exp(s_masked - m), 0.0)
    sm = e / jnp.sum(e, axis=0, keepdims=True)

    tg = tgT_ref[...]
    cl = claT_ref[...]
    pad = jnp.float32(180.0)
    cla0 = jnp.where(cl != 0, pad, tg)
    cla1 = jnp.where(cl != 1, pad, tg)
    cla2 = jnp.where(cl != 2, pad, tg)

    inv_s0 = 1.0 / sigma0
    inv_s11 = 1.0 / sigma1_1
    inv_s12 = 1.0 / sigma1_2
    inv_s2 = 1.0 / sigma2
    pg = (fs0 * inv_s0 * _INV_SQRT_2PI) * jnp.exp(
        -0.5 * ((cla0 - mu0) * inv_s0) ** 2)
    pg = pg + (fs1_1 * inv_s11 * _INV_SQRT_2PI) * jnp.exp(
        -0.5 * ((cla1 - mu1_1) * inv_s11) ** 2)
    pg = pg + (fs1_2 * inv_s12 * _INV_SQRT_2PI) * jnp.exp(
        -0.5 * ((cla1 - mu1_2) * inv_s12) ** 2)
    pg = pg + (fs2_1 * inv_s2 * _INV_SQRT_2PI) * jnp.exp(
        -0.5 * ((cla2 - mu2) * inv_s2) ** 2)
    pg = pg + fs2_2 * jnp.exp(p2 * jnp.log(cla2))

    pg_full = jnp.concatenate([jnp.zeros((1, _BB), jnp.float32), pg], axis=0)
    wT_ref[...] = (1.0 - w_p_g) * sm + w_p_g * pg_full


def _tc_weights(xT, idsT, tgT, claT, wq, wk, wv, bq2, bk2, th):
    grid = (B // _BB,)
    return pl.pallas_call(
        _tc_weights_body,
        grid=grid,
        in_specs=[
            pl.BlockSpec((S, D, _BB), lambda i: (0, 0, i)),
            pl.BlockSpec((S, _BB), lambda i: (0, i)),
            pl.BlockSpec((S - 1, _BB), lambda i: (0, i)),
            pl.BlockSpec((S - 1, _BB), lambda i: (0, i)),
            pl.BlockSpec((D, D), lambda i: (0, 0)),
            pl.BlockSpec((D, D), lambda i: (0, 0)),
            pl.BlockSpec((1, D), lambda i: (0, 0)),
            pl.BlockSpec((1, D), lambda i: (0, 0)),
            pl.BlockSpec((1, D), lambda i: (0, 0)),
            pl.BlockSpec(memory_space=pltpu.SMEM),
        ],
        out_specs=pl.BlockSpec((S, _BB), lambda i: (0, i)),
        out_shape=jax.ShapeDtypeStruct((S, B), jnp.float32),
        scratch_shapes=[pltpu.VMEM((S, _BB), jnp.float32)],
    )(xT, idsT, tgT, claT, wq, wk, wv, bq2, bk2, th)


def _sc_scatter_body(idsT_hbm, wT_hbm, out_hbm, ids_v, w_v, acc, sem):
    wid = lax.axis_index("s") * _NC + lax.axis_index("c")
    col0 = wid * _RPW
    blk0 = (wid // 4) * 128
    my0 = (wid % 4) * _RPW
    c1 = pltpu.async_copy(idsT_hbm.at[:, pl.ds(blk0, 128)], ids_v, sem)
    c2 = pltpu.async_copy(wT_hbm.at[:, pl.ds(blk0, 128)], w_v, sem)

    zeros16 = jnp.zeros((_L,), jnp.float32)

    def _zero(i, c):
        r = i // 8
        off = (i % 8) * 128
        for u in range(8):
            acc[r, pl.ds(off + u * _L, _L)] = zeros16
        return c

    lax.fori_loop(0, _RPW * _VPAD // (8 * _L), _zero, 0)
    c1.wait()
    c2.wait()

    lane = lax.iota(jnp.int32, _L)
    for g in range(_GROUPS):
        rows = lane + g * _L

        def _scat(s, c, rows=rows, g=g):
            ids16 = ids_v[s, pl.ds(my0 + g * _L, _L)]
            w16 = w_v[s, pl.ds(my0 + g * _L, _L)]
            plsc.addupdate_scatter(acc, [rows, ids16], w16)
            return c

        lax.fori_loop(0, S, _scat, 0)

    pltpu.sync_copy(acc, out_hbm.at[pl.ds(col0, _RPW), :])


@functools.cache
def _sc_scatter():
    return pl.kernel(
        _sc_scatter_body,
        out_type=jax.ShapeDtypeStruct((B, _VPAD), jnp.float32),
        mesh=plsc.VectorSubcoreMesh(core_axis_name="c", subcore_axis_name="s",
                                    num_cores=_NC, num_subcores=_NS),
        scratch_types=[
            pltpu.VMEM((S, 128), jnp.int32),
            pltpu.VMEM((S, 128), jnp.float32),
            pltpu.VMEM((_RPW, _VPAD), jnp.float32),
            pltpu.SemaphoreType.DMA,
        ],
        compiler_params=pltpu.CompilerParams(needs_layout_passes=False),
    )


def kernel(x, x_ids, time_gap, cla, Wq, bq, Wk, bk, Wv, bv, theta):
    ids = x_ids.astype(jnp.int32)
    xT = jnp.transpose(x, (1, 2, 0))
    idsT = ids.T
    tgT = time_gap.astype(jnp.float32).T
    claT = cla.astype(jnp.int32).T
    th = jnp.stack([
        theta["fs0"], theta["fs1_1"], theta["fs1_2"], theta["fs2_1"],
        theta["fs2_2"], theta["w_p_g"], theta["mu0"], theta["sigma0"],
        theta["mu1_1"], theta["sigma1_1"], theta["mu1_2"], theta["sigma1_2"],
        theta["mu2"], theta["sigma2"], theta["p2"], bv[0],
    ]).astype(jnp.float32)

    wT = _tc_weights(xT, idsT, tgT, claT, Wq, Wk, Wv, bq[None, :],
                     bk[None, :], th)
    return _sc_scatter()(idsT, wT)[:, :V]
